# baseline (speedup 1.0000x reference)
import jax
import jax.numpy as jnp
from jax.experimental import pallas as pl

PASS = 8
FLOW = 16
TRANS = FLOW - PASS
NBINS = 64
HID = 64
BLOCK = 8192
BIG = 3.0e38


def _coupling_kernel(yT_ref, W1T_ref, b1_ref, W2T_ref, b2_ref, W3T_ref,
                     tric_ref, out_ref):
    yT = yT_ref[...]
    yAT = yT[:PASS, :]

    h = jnp.maximum(
        jnp.dot(W1T_ref[...], yAT, preferred_element_type=jnp.float32)
        + b1_ref[...], 0.0)
    h = jnp.maximum(
        jnp.dot(W2T_ref[...], h, preferred_element_type=jnp.float32)
        + b2_ref[...], 0.0)
    inv_prod = yT[FLOW:FLOW + 1, :]
    for t in range(TRANS):
        Et = jnp.exp(jnp.dot(W3T_ref[t * NBINS:(t + 1) * NBINS, :], h,
                             preferred_element_type=jnp.float32))
        Qs = jnp.dot(tric_ref[t * NBINS:(t + 1) * NBINS, :], Et,
                     preferred_element_type=jnp.float32)
        Qnorm = Qs[NBINS - 1:NBINS, :]
        u = yT[PASS + t:PASS + t + 1, :] * Qnorm
        piv = Qs[NBINS // 2 - 1:NBINS // 2, :]
        go = piv < u
        seg = jnp.where(go, Qs[NBINS // 2:NBINS - 1, :],
                        Qs[:NBINS // 2 - 1, :])
        base = jnp.where(go, float(NBINS // 2), 0.0)
        flo = jnp.where(go, piv, 0.0)
        cap = jnp.where(go, Qnorm, piv)
        n = NBINS // 2 - 1
        while n > 1:
            m = n // 2
            piv = seg[m:m + 1, :]
            go = piv < u
            base = base + jnp.where(go, float(m + 1), 0.0)
            flo = jnp.where(go, piv, flo)
            cap = jnp.where(go, cap, piv)
            seg = jnp.where(go, seg[m + 1:n, :], seg[:m, :])
            n = m
        go = seg < u
        k = base + jnp.where(go, 1.0, 0.0)
        offset = jnp.where(go, seg, flo)
        nxt = jnp.where(go, cap, seg)
        rs = 1.0 / ((nxt - offset) * float(NBINS))
        out_ref[t:t + 1, :] = (u - offset) * rs + k * (1.0 / NBINS)
        inv_prod = inv_prod * (Qnorm * rs)

    out_ref[TRANS:TRANS + 1, :] = inv_prod


def kernel(y, W1, b1, W2, b2, W3, b3):
    B = y.shape[0]
    grid = (B // BLOCK,)
    tri = (jnp.arange(NBINS)[:, None] >= jnp.arange(NBINS)[None, :]
           ).astype(jnp.float32)
    tric = (tri[None, :, :] * jnp.exp(b3).reshape(TRANS, 1, NBINS)
            ).reshape(TRANS * NBINS, NBINS)
    o9 = pl.pallas_call(
        _coupling_kernel,
        grid=grid,
        in_specs=[
            pl.BlockSpec((FLOW + 1, BLOCK), lambda i: (0, i)),
            pl.BlockSpec((HID, PASS), lambda i: (0, 0)),
            pl.BlockSpec((HID, 1), lambda i: (0, 0)),
            pl.BlockSpec((HID, HID), lambda i: (0, 0)),
            pl.BlockSpec((HID, 1), lambda i: (0, 0)),
            pl.BlockSpec((TRANS * NBINS, HID), lambda i: (0, 0)),
            pl.BlockSpec((TRANS * NBINS, NBINS), lambda i: (0, 0)),
        ],
        out_specs=pl.BlockSpec((TRANS + 1, BLOCK), lambda i: (0, i)),
        out_shape=jax.ShapeDtypeStruct((TRANS + 1, B), jnp.float32),
    )(y.T, W1.T, b1[:, None], W2.T, b2[:, None], W3.T, tric)
    return jnp.concatenate([y[:, :PASS], o9.T], axis=1)


# speedup vs baseline: 1.0365x; 1.0365x over previous
import jax
import jax.numpy as jnp
from jax.experimental import pallas as pl

PASS = 8
FLOW = 16
TRANS = FLOW - PASS
NBINS = 64
HID = 64
BLOCK = 8192
BIG = 3.0e38


def _coupling_kernel(yT_ref, W1T_ref, b1_ref, W2T_ref, b2_ref, W3T_ref,
                     tric_ref, out_ref):
    yT = yT_ref[...]
    yAT = yT[:PASS, :]

    h = jnp.maximum(
        jnp.dot(W1T_ref[...], yAT, preferred_element_type=jnp.float32)
        + b1_ref[...], 0.0)
    h = jnp.maximum(
        jnp.dot(W2T_ref[...], h, preferred_element_type=jnp.float32)
        + b2_ref[...], 0.0)
    inv_prod = yT[FLOW:FLOW + 1, :]
    for t in range(TRANS):
        Et = jnp.exp(jnp.dot(W3T_ref[t * NBINS:(t + 1) * NBINS, :], h,
                             preferred_element_type=jnp.float32))
        Qs = jnp.dot(tric_ref[t * NBINS:(t + 1) * NBINS, :], Et,
                     preferred_element_type=jnp.float32)
        Qnorm = Qs[NBINS - 1:NBINS, :]
        u = yT[PASS + t:PASS + t + 1, :] * Qnorm
        piv = Qs[NBINS // 2 - 1:NBINS // 2, :]
        go = piv < u
        seg = jnp.where(go, Qs[NBINS // 2:NBINS - 1, :],
                        Qs[:NBINS // 2 - 1, :])
        base = jnp.where(go, float(NBINS // 2), 0.0)
        flo = jnp.where(go, piv, 0.0)
        cap = jnp.where(go, Qnorm, piv)
        piv = seg[15:16, :]
        go = piv < u
        base = base + jnp.where(go, 16.0, 0.0)
        flo = jnp.where(go, piv, flo)
        cap = jnp.where(go, cap, piv)
        seg = jnp.where(go, seg[16:31, :], seg[:15, :])
        lt = seg < u
        k = base + jnp.sum(lt.astype(jnp.float32), axis=0, keepdims=True)
        offset = jnp.maximum(
            jnp.max(jnp.where(lt, seg, 0.0), axis=0, keepdims=True), flo)
        nxt = jnp.minimum(
            jnp.min(jnp.where(lt, BIG, seg), axis=0, keepdims=True), cap)
        rs = 1.0 / ((nxt - offset) * float(NBINS))
        out_ref[t:t + 1, :] = (u - offset) * rs + k * (1.0 / NBINS)
        inv_prod = inv_prod * (Qnorm * rs)

    out_ref[TRANS:TRANS + 1, :] = inv_prod


def kernel(y, W1, b1, W2, b2, W3, b3):
    B = y.shape[0]
    grid = (B // BLOCK,)
    tri = (jnp.arange(NBINS)[:, None] >= jnp.arange(NBINS)[None, :]
           ).astype(jnp.float32)
    tric = (tri[None, :, :] * jnp.exp(b3).reshape(TRANS, 1, NBINS)
            ).reshape(TRANS * NBINS, NBINS)
    o9 = pl.pallas_call(
        _coupling_kernel,
        grid=grid,
        in_specs=[
            pl.BlockSpec((FLOW + 1, BLOCK), lambda i: (0, i)),
            pl.BlockSpec((HID, PASS), lambda i: (0, 0)),
            pl.BlockSpec((HID, 1), lambda i: (0, 0)),
            pl.BlockSpec((HID, HID), lambda i: (0, 0)),
            pl.BlockSpec((HID, 1), lambda i: (0, 0)),
            pl.BlockSpec((TRANS * NBINS, HID), lambda i: (0, 0)),
            pl.BlockSpec((TRANS * NBINS, NBINS), lambda i: (0, 0)),
        ],
        out_specs=pl.BlockSpec((TRANS + 1, BLOCK), lambda i: (0, i)),
        out_shape=jax.ShapeDtypeStruct((TRANS + 1, B), jnp.float32),
    )(y.T, W1.T, b1[:, None], W2.T, b2[:, None], W3.T, tric)
    return jnp.concatenate([y[:, :PASS], o9.T], axis=1)


# speedup vs baseline: 1.0766x; 1.0387x over previous
import jax
import jax.numpy as jnp
from jax.experimental import pallas as pl

PASS = 8
FLOW = 16
TRANS = FLOW - PASS
NBINS = 64
HID = 64
BLOCK = 8192
BIG = 3.0e38


def _coupling_kernel(yT_ref, W1T_ref, b1_ref, W2T_ref, b2_ref, W3T_ref,
                     tric_ref, out_ref):
    yT = yT_ref[...]
    yAT = yT[:PASS, :]

    h = jnp.maximum(
        jnp.dot(W1T_ref[...], yAT, preferred_element_type=jnp.float32)
        + b1_ref[...], 0.0)
    h = jnp.maximum(
        jnp.dot(W2T_ref[...], h, preferred_element_type=jnp.float32)
        + b2_ref[...], 0.0)
    inv_prod = yT[FLOW:FLOW + 1, :]
    for t in range(TRANS):
        Et = jnp.exp2(jnp.dot(W3T_ref[t * NBINS:(t + 1) * NBINS, :], h,
                              preferred_element_type=jnp.float32))
        Qs = jnp.dot(tric_ref[t * NBINS:(t + 1) * NBINS, :], Et,
                     preferred_element_type=jnp.float32)
        Qnorm = Qs[NBINS - 1:NBINS, :]
        u = yT[PASS + t:PASS + t + 1, :] * Qnorm
        piv = Qs[NBINS // 2 - 1:NBINS // 2, :]
        go = piv < u
        seg = jnp.where(go, Qs[NBINS // 2:NBINS - 1, :],
                        Qs[:NBINS // 2 - 1, :])
        base = jnp.where(go, float(NBINS // 2), 0.0)
        flo = jnp.where(go, piv, 0.0)
        cap = jnp.where(go, Qnorm, piv)
        piv = seg[15:16, :]
        go = piv < u
        base = base + jnp.where(go, 16.0, 0.0)
        flo = jnp.where(go, piv, flo)
        cap = jnp.where(go, cap, piv)
        seg = jnp.where(go, seg[16:31, :], seg[:15, :])
        lt = seg < u
        k = base + jnp.sum(lt.astype(jnp.float32), axis=0, keepdims=True)
        offset = jnp.maximum(
            jnp.max(jnp.where(lt, seg, 0.0), axis=0, keepdims=True), flo)
        nxt = jnp.minimum(
            jnp.min(jnp.where(lt, BIG, seg), axis=0, keepdims=True), cap)
        rs = 1.0 / ((nxt - offset) * float(NBINS))
        out_ref[t:t + 1, :] = (u - offset) * rs + k * (1.0 / NBINS)
        inv_prod = inv_prod * (Qnorm * rs)

    out_ref[TRANS:TRANS + 1, :] = inv_prod


def kernel(y, W1, b1, W2, b2, W3, b3):
    B = y.shape[0]
    grid = (B // BLOCK,)
    tri = (jnp.arange(NBINS)[:, None] >= jnp.arange(NBINS)[None, :]
           ).astype(jnp.float32)
    tric = (tri[None, :, :] * jnp.exp(b3).reshape(TRANS, 1, NBINS)
            ).reshape(TRANS * NBINS, NBINS)
    o9 = pl.pallas_call(
        _coupling_kernel,
        grid=grid,
        in_specs=[
            pl.BlockSpec((FLOW + 1, BLOCK), lambda i: (0, i)),
            pl.BlockSpec((HID, PASS), lambda i: (0, 0)),
            pl.BlockSpec((HID, 1), lambda i: (0, 0)),
            pl.BlockSpec((HID, HID), lambda i: (0, 0)),
            pl.BlockSpec((HID, 1), lambda i: (0, 0)),
            pl.BlockSpec((TRANS * NBINS, HID), lambda i: (0, 0)),
            pl.BlockSpec((TRANS * NBINS, NBINS), lambda i: (0, 0)),
        ],
        out_specs=pl.BlockSpec((TRANS + 1, BLOCK), lambda i: (0, i)),
        out_shape=jax.ShapeDtypeStruct((TRANS + 1, B), jnp.float32),
    )(y.T, W1.T, b1[:, None], W2.T, b2[:, None],
      W3.T * 1.4426950408889634, tric)
    return jnp.concatenate([y[:, :PASS], o9.T], axis=1)
